# baseline (speedup 1.0000x reference)
import functools

import jax
import jax.numpy as jnp
from jax import lax
from jax.experimental import pallas as pl
from jax.experimental.pallas import tpu as pltpu
from jax.experimental.pallas import tpu_sc as plsc

N_VOCAB = 1000000
N_EMBED = 64
N_ROWS = 16384
N_IDX = 50

NC = 2
NS = 16
NW = NC * NS

PER_W = N_ROWS // NW
T = 8
N_CHUNKS = PER_W // T

_mesh = plsc.VectorSubcoreMesh(
    core_axis_name="c", subcore_axis_name="s", num_cores=NC, num_subcores=NS
)


@functools.partial(
    pl.kernel,
    mesh=_mesh,
    compiler_params=pltpu.CompilerParams(use_tc_tiling_on_sc=False),
    out_type=jax.ShapeDtypeStruct((N_ROWS, N_IDX, N_EMBED), jnp.float32),
    scratch_types=[
        pltpu.VMEM((PER_W, N_IDX), jnp.int32),
        pltpu.VMEM((T, N_IDX, N_EMBED), jnp.float32),
        pltpu.VMEM((T, N_IDX, N_EMBED), jnp.float32),
        pltpu.SemaphoreType.DMA,
        pltpu.SemaphoreType.DMA,
        pltpu.SemaphoreType.DMA,
        pltpu.SemaphoreType.DMA,
    ],
)
def _emb_lookup(idx_hbm, table_hbm, out_hbm, idx_all, rows0, rows1,
                semg0, semg1, sems0, sems1):
    wid = lax.axis_index("s") * NC + lax.axis_index("c")
    tbase = wid * PER_W

    pltpu.sync_copy(idx_hbm.at[pl.ds(tbase, PER_W)], idx_all)

    rows = (rows0, rows1)
    semg = (semg0, semg1)
    sems = (sems0, sems1)

    def fire_gathers(c, b):
        for t in range(T):
            pltpu.async_copy(
                table_hbm.at[idx_all.at[c * T + t]], rows[b].at[t], semg[b]
            )

    def drain_gathers(b):
        pltpu.make_async_copy(out_hbm.at[pl.ds(0, T)], rows[b], semg[b]).wait()

    def fire_store(c, b):
        pltpu.async_copy(rows[b], out_hbm.at[pl.ds(tbase + c * T, T)], sems[b])

    def drain_store(b):
        pltpu.make_async_copy(rows[b], out_hbm.at[pl.ds(0, T)], sems[b]).wait()

    fire_gathers(0, 0)

    def pair_body(p, carry):
        c0 = 2 * p

        @pl.when(p >= 1)
        def _():
            drain_store(1)

        fire_gathers(c0 + 1, 1)
        drain_gathers(0)
        fire_store(c0, 0)

        @pl.when(p < N_CHUNKS // 2 - 1)
        def _():
            drain_store(0)
            fire_gathers(c0 + 2, 0)

        drain_gathers(1)
        fire_store(c0 + 1, 1)
        return carry

    lax.fori_loop(0, N_CHUNKS // 2, pair_body, 0)

    drain_store(0)
    drain_store(1)


def kernel(x, weight):
    return _emb_lookup(x.astype(jnp.int32), weight)
